# baseline (speedup 1.0000x reference)
import functools

import jax
import jax.numpy as jnp
from jax import lax
from jax.experimental import pallas as pl
from jax.experimental.pallas import tpu as pltpu
from jax.experimental.pallas import tpu_sc as plsc

_B = 16384
_D = 32
_NC, _NS = 2, 16
_NW = _NC * _NS
_BPW = _B // _NW
_CHUNK = 128
_NCHUNK = _BPW // _CHUNK

_BLK = 2048


def _sc_gather4(u_idx3, i_idx3, t_ug, t_ig, t_um, t_im):
    mesh = plsc.VectorSubcoreMesh(core_axis_name="c", subcore_axis_name="s")
    out4 = jax.ShapeDtypeStruct((_NW, _NCHUNK, _CHUNK, _D), jnp.float32)

    @functools.partial(
        pl.kernel,
        mesh=mesh,
        out_type=[out4, out4, out4, out4],
        compiler_params=pltpu.CompilerParams(use_tc_tiling_on_sc=False),
        scratch_types=[
            pltpu.VMEM((_NCHUNK, _CHUNK), jnp.int32),
            pltpu.VMEM((_NCHUNK, _CHUNK), jnp.int32),
            pltpu.VMEM((_NCHUNK, _CHUNK, _D), jnp.float32),
            pltpu.VMEM((_NCHUNK, _CHUNK, _D), jnp.float32),
            pltpu.VMEM((_NCHUNK, _CHUNK, _D), jnp.float32),
            pltpu.VMEM((_NCHUNK, _CHUNK, _D), jnp.float32),
            pltpu.SemaphoreType.DMA,
        ],
    )
    def k(uidx_hbm, iidx_hbm, ug_hbm, ig_hbm, um_hbm, im_hbm,
          o_ug, o_ig, o_um, o_im,
          uix_v, iix_v, r_ug, r_ig, r_um, r_im, sem):
        wid = lax.axis_index("s") * _NC + lax.axis_index("c")
        pltpu.sync_copy(uidx_hbm.at[wid], uix_v)
        pltpu.sync_copy(iidx_hbm.at[wid], iix_v)
        copies = []
        for c in range(_NCHUNK):
            copies.append(pltpu.async_copy(ug_hbm.at[uix_v.at[c]], r_ug.at[c], sem))
            copies.append(pltpu.async_copy(ig_hbm.at[iix_v.at[c]], r_ig.at[c], sem))
            copies.append(pltpu.async_copy(um_hbm.at[uix_v.at[c]], r_um.at[c], sem))
            copies.append(pltpu.async_copy(im_hbm.at[iix_v.at[c]], r_im.at[c], sem))
        for cp in copies:
            cp.wait()
        pltpu.sync_copy(r_ug, o_ug.at[wid])
        pltpu.sync_copy(r_ig, o_ig.at[wid])
        pltpu.sync_copy(r_um, o_um.at[wid])
        pltpu.sync_copy(r_im, o_im.at[wid])

    return k(u_idx3, i_idx3, t_ug, t_ig, t_um, t_im)


def _mlp_body(ug_ref, ig_ref, um_ref, im_ref, w1a_ref, w1b_ref, b1_ref,
              w2_ref, b2_ref, wpa_ref, wpb_ref, bp_ref, o_ref):
    h1 = jnp.dot(um_ref[...], w1a_ref[...], preferred_element_type=jnp.float32)
    h1 += jnp.dot(im_ref[...], w1b_ref[...], preferred_element_type=jnp.float32)
    h1 = jnp.maximum(h1 + b1_ref[...], 0.0)
    h2 = jnp.dot(h1, w2_ref[...], preferred_element_type=jnp.float32)
    h2 = jnp.maximum(h2 + b2_ref[...], 0.0)
    g = ug_ref[...] * ig_ref[...]
    p = (jnp.sum(g * wpa_ref[...], axis=1, keepdims=True)
         + jnp.sum(h2 * wpb_ref[...], axis=1, keepdims=True)
         + bp_ref[...])
    o_ref[...] = jax.nn.sigmoid(p)


def _tc_mlp(ug, ig, um, im, w1a, w1b, b1r, w2, b2r, wpa, wpb, bpr):
    emb_spec = pl.BlockSpec((_BLK, _D), lambda i: (i, 0))

    def full(shape):
        return pl.BlockSpec(shape, lambda i: (0, 0))

    return pl.pallas_call(
        _mlp_body,
        grid=(_B // _BLK,),
        in_specs=[
            emb_spec, emb_spec, emb_spec, emb_spec,
            full((_D, 32)), full((_D, 32)), full((1, 32)),
            full((32, 16)), full((1, 16)),
            full((1, _D)), full((1, 16)), full((1, 1)),
        ],
        out_specs=pl.BlockSpec((_BLK, 1), lambda i: (i, 0)),
        out_shape=jax.ShapeDtypeStruct((_B, 1), jnp.float32),
    )(ug, ig, um, im, w1a, w1b, b1r, w2, b2r, wpa, wpb, bpr)


def kernel(user_indices, item_indices, embed_user_GMF, embed_item_GMF,
           embed_user_MLP, embed_item_MLP, W1, b1, W2, b2, Wp, bp):
    u3 = user_indices.astype(jnp.int32).reshape(_NW, _NCHUNK, _CHUNK)
    i3 = item_indices.astype(jnp.int32).reshape(_NW, _NCHUNK, _CHUNK)
    ug, ig, um, im = _sc_gather4(u3, i3, embed_user_GMF, embed_item_GMF,
                                 embed_user_MLP, embed_item_MLP)
    ug = ug.reshape(_B, _D)
    ig = ig.reshape(_B, _D)
    um = um.reshape(_B, _D)
    im = im.reshape(_B, _D)
    w1a, w1b = W1[:_D], W1[_D:]
    wpa = Wp[:_D, 0].reshape(1, _D)
    wpb = Wp[_D:, 0].reshape(1, 16)
    out = _tc_mlp(ug, ig, um, im, w1a, w1b, b1.reshape(1, 32),
                  W2, b2.reshape(1, 16), wpa, wpb, bp.reshape(1, 1))
    return out.reshape(-1)
